# Initial kernel scaffold; baseline (speedup 1.0000x reference)
import jax
import jax.numpy as jnp
from jax.experimental import pallas as pl


def kernel(timestamps, table):
    raise NotImplementedError("write your pallas kernel here")


# speedup vs baseline: 3.8000x; 3.8000x over previous
import functools

import jax
import jax.numpy as jnp
from jax.experimental import pallas as pl
from jax.experimental.pallas import tpu as pltpu
from jax.experimental.pallas import tpu_sc as plsc

_ROWS = 50000
_DIM = 64
_DELTAT = 1.0 / 50000.0
_LANES = 16
_W = 128


def kernel(timestamps, table):
    n_rows, n_cols = timestamps.shape
    n = n_rows * n_cols
    ts_flat = timestamps.reshape(1, n)
    mesh = plsc.VectorSubcoreMesh(core_axis_name="c", subcore_axis_name="s")

    @functools.partial(
        pl.kernel,
        out_type=jax.ShapeDtypeStruct((n, _DIM), jnp.float32),
        mesh=mesh,
        scratch_types=[pltpu.VMEM((_W,), jnp.int32)],
        compiler_params=pltpu.CompilerParams(use_tc_tiling_on_sc=False),
    )
    def sc_gather(ts_hbm, table_hbm, out_hbm, idx_v):
        def body(ts_vmem, o_vmem):
            @pl.loop(0, _W, step=_LANES)
            def _(i):
                t = ts_vmem[0, pl.ds(i, _LANES)]
                ix = (t / jnp.float32(_DELTAT)).astype(jnp.int32)
                idx_v[pl.ds(i, _LANES)] = jnp.minimum(
                    jnp.maximum(ix, 0), _ROWS - 1
                )

            pltpu.sync_copy(table_hbm.at[idx_v], o_vmem)

        pltpu.emit_pipeline(
            body,
            grid=(n // _W,),
            in_specs=[pl.BlockSpec((1, _W), index_map=lambda i: (0, i))],
            out_specs=[pl.BlockSpec((_W, _DIM), index_map=lambda i: (i, 0))],
            core_axis_name=("c", "s"),
            dimension_semantics=(pltpu.PARALLEL,),
        )(ts_hbm, out_hbm)

    out = sc_gather(ts_flat, table)
    return out.reshape(n_rows, n_cols, _DIM)
